# baseline (speedup 1.0000x reference)
import functools

import jax
import jax.numpy as jnp
from jax import lax
from jax.experimental import pallas as pl
from jax.experimental.pallas import tpu as pltpu
from jax.experimental.pallas import tpu_sc as plsc

N = 50000
V = 5143
W = 8
LD = 32
K = 9

TN = 200
VPAD = 5248
NW = 32
GROWS = N * K
CHUNK = 128
NCH = 110
GPAD = NW * NCH * CHUNK
PB = TN * K


def _knn_body(means_ref, ct_ref, c2_ref, vpt_ref, cvpt_ref, lat_ref, can_ref,
              idx_ref, tab_ref):
    i = pl.program_id(0)

    @pl.when(i == 0)
    def _():
        tab_ref[:, 0:24] = vpt_ref[...] - cvpt_ref[...]
        tab_ref[:, 24:56] = lat_ref[...]
        tab_ref[:, 56:59] = can_ref[...]
        tab_ref[:, 59:128] = jnp.zeros((V, 69), jnp.float32)

    m = means_ref[...]
    cross = lax.dot_general(m, ct_ref[...], (((1,), (0,)), ((), ())),
                            preferred_element_type=jnp.float32)
    s = c2_ref[...] - 2.0 * cross
    iota = lax.broadcasted_iota(jnp.int32, (TN, VPAD), 1)
    inf = jnp.float32(jnp.inf)
    for k in range(K):
        idxk = jnp.argmin(s, axis=1).astype(jnp.int32)[:, None]
        idx_ref[:, k:k + 1] = idxk
        if k + 1 < K:
            s = jnp.where(iota == idxk, inf, s)


def _tc_knn(means, ct, c2, vpt, cvpt, latents, canon0):
    grid = N // TN
    return pl.pallas_call(
        _knn_body,
        grid=(grid,),
        in_specs=[
            pl.BlockSpec((TN, 3), lambda i: (i, 0)),
            pl.BlockSpec((3, VPAD), lambda i: (0, 0)),
            pl.BlockSpec((1, VPAD), lambda i: (0, 0)),
            pl.BlockSpec((V, 24), lambda i: (0, 0)),
            pl.BlockSpec((V, 24), lambda i: (0, 0)),
            pl.BlockSpec((V, LD), lambda i: (0, 0)),
            pl.BlockSpec((V, 3), lambda i: (0, 0)),
        ],
        out_specs=[
            pl.BlockSpec((TN, K), lambda i: (i, 0)),
            pl.BlockSpec((V, 128), lambda i: (0, 0)),
        ],
        out_shape=[
            jax.ShapeDtypeStruct((N, K), jnp.int32),
            jax.ShapeDtypeStruct((V, 128), jnp.float32),
        ],
    )(means, ct, c2, vpt, cvpt, latents, canon0)


def _sc_gather(idx3, tab):
    mesh = plsc.VectorSubcoreMesh(core_axis_name="c", subcore_axis_name="s")

    @functools.partial(
        pl.kernel,
        out_type=jax.ShapeDtypeStruct((NW, NCH, CHUNK, 128), jnp.float32),
        mesh=mesh,
        scratch_types=[
            pltpu.VMEM((NCH, CHUNK), jnp.int32),
            pltpu.VMEM((CHUNK, 128), jnp.float32),
            pltpu.VMEM((CHUNK, 128), jnp.float32),
            pltpu.SemaphoreType.DMA,
            pltpu.SemaphoreType.DMA,
        ],
    )
    def k(idx_hbm, tab_hbm, got_hbm, idx_v, va, vb, sem_a, sem_b):
        wid = lax.axis_index("s") * 2 + lax.axis_index("c")
        pltpu.sync_copy(idx_hbm.at[wid], idx_v)

        pltpu.async_copy(tab_hbm.at[idx_v.at[0]], va, sem_a)

        @pl.loop(0, NCH, step=2)
        def _(c):
            cb = pltpu.async_copy(tab_hbm.at[idx_v.at[c + 1]], vb, sem_b)
            pltpu.make_async_copy(tab_hbm.at[idx_v.at[c]], va, sem_a).wait()
            pltpu.sync_copy(va, got_hbm.at[wid, c])

            @pl.when(c + 2 < NCH)
            def _():
                pltpu.async_copy(tab_hbm.at[idx_v.at[c + 2]], va, sem_a)

            cb.wait()
            pltpu.sync_copy(vb, got_hbm.at[wid, c + 1])

    return k(idx3, tab)


def _post_body(g_ref, m_ref, mo_ref, la_ref, di_ref):
    g = g_ref[...]
    mo_ref[...] = g[:, 0:24]
    la_ref[...] = g[:, 24:56]
    m = m_ref[...]
    m9 = jnp.broadcast_to(m[:, None, :], (TN, K, 3)).reshape(PB, 3)
    dx = g[:, 56:57] - m9[:, 0:1]
    dy = g[:, 57:58] - m9[:, 1:2]
    dz = g[:, 58:59] - m9[:, 2:3]
    di_ref[...] = jnp.sqrt(dx * dx + dy * dy + dz * dz + 1e-12)


def _tc_post(flat, means):
    grid = GROWS // PB
    return pl.pallas_call(
        _post_body,
        grid=(grid,),
        in_specs=[
            pl.BlockSpec((PB, 128), lambda i: (i, 0)),
            pl.BlockSpec((TN, 3), lambda i: (i, 0)),
        ],
        out_specs=[
            pl.BlockSpec((PB, 24), lambda i: (i, 0)),
            pl.BlockSpec((PB, LD), lambda i: (i, 0)),
            pl.BlockSpec((PB, 1), lambda i: (i, 0)),
        ],
        out_shape=[
            jax.ShapeDtypeStruct((GROWS, 24), jnp.float32),
            jax.ShapeDtypeStruct((GROWS, LD), jnp.float32),
            jax.ShapeDtypeStruct((GROWS, 1), jnp.float32),
        ],
    )(flat, means)


def kernel(means, vertex_positions, canonical_vertex_positions, latents_table):
    canon0 = canonical_vertex_positions[0]
    ct = jnp.pad(canon0.T, ((0, 0), (0, VPAD - V)))
    c2 = jnp.sum(canon0 * canon0, axis=1)
    c2 = jnp.pad(c2, (0, VPAD - V), constant_values=1e30)[None, :]
    vpt = jnp.transpose(vertex_positions, (1, 0, 2)).reshape(V, W * 3)
    cvpt = jnp.transpose(canonical_vertex_positions, (1, 0, 2)).reshape(V, W * 3)

    idx, tab = _tc_knn(means, ct, c2, vpt, cvpt, latents_table, canon0)

    idx3 = jnp.pad(idx.reshape(-1), (0, GPAD - GROWS)).reshape(NW, NCH, CHUNK)
    got = _sc_gather(idx3, tab)

    flat = got.reshape(GPAD, 128)
    mo, la, di = _tc_post(flat, means)

    knn_motion = mo.reshape(N, K, W * 3)
    knn_latents = la.reshape(N, K, LD)
    knn_dists = di.reshape(N, K, 1)
    return knn_motion, knn_latents, knn_dists
